# Initial kernel scaffold; baseline (speedup 1.0000x reference)
import jax
import jax.numpy as jnp
from jax.experimental import pallas as pl


def kernel(x, edge_index, batch, W1, b1, bn1_g, bn1_b, W2, b2, bn2_g, bn2_b, fc1_W, fc1_b, bn3_g, bn3_b, fc2_W, fc2_b, fc3_W, fc3_b):
    raise NotImplementedError("write your pallas kernel here")


# speedup vs baseline: 12.9688x; 12.9688x over previous
import functools

import jax
import jax.numpy as jnp
from jax import lax
from jax.experimental import pallas as pl
from jax.experimental.pallas import tpu as pltpu
from jax.experimental.pallas import tpu_sc as plsc

NN = 50000
NE = 800000
NG = 512
NF = 75

NC = 2
NS = 16
NP = 50048
TPS = NP // NS
CH = 128
NCH = 6
SUPE = NCH * CH
NSUP = 66
EPT = NSUP * SUPE
EP = EPT * NS
ROWS_PT = EPT // CH

BLK = 2000
NBLK = NN // BLK
EPS = 1e-5

_mesh = plsc.VectorSubcoreMesh(core_axis_name="c", subcore_axis_name="s")


@functools.partial(
    pl.kernel,
    out_type=jax.ShapeDtypeStruct((NC * NP,), jnp.float32),
    mesh=_mesh,
    scratch_types=[
        pltpu.VMEM((CH,), jnp.float32),
        pltpu.VMEM((3, CH), jnp.int32),
        pltpu.VMEM((TPS,), jnp.float32),
        pltpu.VMEM_SHARED((NP,), jnp.float32),
    ],
    compiler_params=pltpu.CompilerParams(use_tc_tiling_on_sc=False),
)
def _deg_kernel(dst2, zeros1, out, ones_v, dstb, stage, hist):
    cid = lax.axis_index("c")
    sid = lax.axis_index("s")
    wid = cid * NS + sid

    def ones_body(i, c):
        ones_v[pl.ds(i * 16, 16)] = jnp.full((16,), 1.0, jnp.float32)
        return c
    lax.fori_loop(0, CH // 16, ones_body, 0)

    pltpu.sync_copy(zeros1, stage)
    pltpu.sync_copy(stage, hist.at[pl.ds(sid * TPS, TPS)])
    plsc.subcore_barrier()

    cpw = EP // (NC * NS) // CH

    def super_body(sup, c):
        pltpu.sync_copy(dst2.at[pl.ds(wid * cpw + sup * 3, 3)], dstb)
        for kk in range(3):
            pltpu.sync_copy(ones_v, hist.at[dstb.at[kk]], add=True)
        return c
    lax.fori_loop(0, cpw // 3, super_body, 0)

    plsc.subcore_barrier()
    pltpu.sync_copy(hist.at[pl.ds(sid * TPS, TPS)], stage)
    pltpu.sync_copy(stage, out.at[pl.ds(cid * NP + sid * TPS, TPS)])


def _make_msg(S):
    @functools.partial(
        pl.kernel,
        out_type=jax.ShapeDtypeStruct((NC * S * NP, 32), jnp.float32),
        mesh=_mesh,
        scratch_types=[
            pltpu.VMEM((SUPE,), jnp.int32),
            pltpu.VMEM((SUPE,), jnp.int32),
            pltpu.VMEM((NCH, CH), jnp.int32),
            pltpu.VMEM((SUPE, 32), jnp.float32),
            pltpu.VMEM_SHARED((NP, 32), jnp.float32),
            pltpu.SemaphoreType.DMA,
            pltpu.SemaphoreType.DMA,
        ],
        compiler_params=pltpu.CompilerParams(use_tc_tiling_on_sc=False),
    )
    def msg(table, srcp, dst2, zeros2, out,
            src_v, gidx, dst_v, rows, acc, sem_g, sem_s):
        cid = lax.axis_index("c")
        sid = lax.axis_index("s")

        wb_chunks = [(0, SUPE), (SUPE, SUPE), (2 * SUPE, SUPE),
                     (3 * SUPE, SUPE), (4 * SUPE, TPS - 4 * SUPE)]

        for j in range(S):
            p = cid * S + j
            base_row = p * NN
            pltpu.sync_copy(zeros2, rows)
            for off, sz in wb_chunks:
                pltpu.sync_copy(rows.at[pl.ds(0, sz)],
                                acc.at[pl.ds(sid * TPS + off, sz)])
            plsc.subcore_barrier()

            def super_body(sup, c):
                ebase = sid * EPT + sup * SUPE
                pltpu.sync_copy(srcp.at[pl.ds(ebase, SUPE)], src_v)
                pltpu.sync_copy(
                    dst2.at[pl.ds(sid * ROWS_PT + sup * NCH, NCH)], dst_v)

                def add_body(i, cc):
                    sl = pl.ds(i * 16, 16)
                    gidx[sl] = src_v[sl] + base_row
                    return cc
                lax.fori_loop(0, SUPE // 16, add_body, 0)

                gd = [pltpu.async_copy(table.at[gidx.at[pl.ds(kk * CH, CH)]],
                                       rows.at[pl.ds(kk * CH, CH)], sem_g)
                      for kk in range(NCH)]
                for d in gd:
                    d.wait()
                sd = [pltpu.async_copy(rows.at[pl.ds(kk * CH, CH)],
                                       acc.at[dst_v.at[kk]], sem_s, add=True)
                      for kk in range(NCH)]
                for d in sd:
                    d.wait()
                return c
            lax.fori_loop(0, NSUP, super_body, 0)

            plsc.subcore_barrier()
            for off, sz in wb_chunks:
                pltpu.sync_copy(acc.at[pl.ds(sid * TPS + off, sz)],
                                rows.at[pl.ds(0, sz)])
                pltpu.sync_copy(rows.at[pl.ds(0, sz)],
                                out.at[pl.ds(p * NP + sid * TPS + off, sz)])
            plsc.subcore_barrier()

    return msg


_msg1 = _make_msg(2)
_msg2 = _make_msg(1)


def _pre_body(x_ref, w_ref, p0_ref, p1_ref, hws_ref, dinv_ref):
    deg = p0_ref[...] + p1_ref[...] + 1.0
    dinv = lax.rsqrt(deg)
    dinv_ref[...] = dinv
    hw = jnp.dot(x_ref[...], w_ref[0], preferred_element_type=jnp.float32)
    hws_ref[0] = hw * dinv


def _pre(x, w1, p0, p1):
    return pl.pallas_call(
        _pre_body,
        grid=(4, NBLK),
        in_specs=[
            pl.BlockSpec((BLK, NF), lambda s, i: (i, 0)),
            pl.BlockSpec((1, NF, 32), lambda s, i: (s, 0, 0)),
            pl.BlockSpec((BLK, 1), lambda s, i: (i, 0)),
            pl.BlockSpec((BLK, 1), lambda s, i: (i, 0)),
        ],
        out_specs=[
            pl.BlockSpec((1, BLK, 32), lambda s, i: (s, i, 0)),
            pl.BlockSpec((BLK, 1), lambda s, i: (i, 0)),
        ],
        out_shape=[
            jax.ShapeDtypeStruct((4, NN, 32), jnp.float32),
            jax.ShapeDtypeStruct((NN, 1), jnp.float32),
        ],
    )(x, w1, p0, p1)


def _make_post(S):
    def body(acc_ref, hws_ref, dinv_ref, b_ref, h_ref, st_ref):
        i = pl.program_id(0)
        dinv = dinv_ref[...].reshape(1, BLK, 1)
        o = jnp.maximum(dinv * (acc_ref[...] + hws_ref[...]) + b_ref[...], 0.0)
        h_ref[...] = o

        @pl.when(i == 0)
        def _():
            st_ref[...] = jnp.zeros_like(st_ref)

        s0 = jnp.sum(o, axis=1, keepdims=True)
        s1 = jnp.sum(o * o, axis=1, keepdims=True)
        pad = jnp.zeros((S, 6, 32), jnp.float32)
        st_ref[...] += jnp.concatenate([s0, s1, pad], axis=1)

    def call(acc, hws, dinv, br):
        return pl.pallas_call(
            body,
            grid=(NBLK,),
            in_specs=[
                pl.BlockSpec((S, BLK, 32), lambda i: (0, i, 0)),
                pl.BlockSpec((S, BLK, 32), lambda i: (0, i, 0)),
                pl.BlockSpec((BLK, 1), lambda i: (i, 0)),
                pl.BlockSpec((S, 1, 32), lambda i: (0, 0, 0)),
            ],
            out_specs=[
                pl.BlockSpec((S, BLK, 32), lambda i: (0, i, 0)),
                pl.BlockSpec((S, 8, 32), lambda i: (0, 0, 0)),
            ],
            out_shape=[
                jax.ShapeDtypeStruct((S, NN, 32), jnp.float32),
                jax.ShapeDtypeStruct((S, 8, 32), jnp.float32),
            ],
        )(acc, hws, dinv, br)

    return call


_post1 = _make_post(4)
_post2 = _make_post(2)


def _mid_body(h_ref, st_ref, g_ref, b_ref, w2_ref, dinv_ref, out_ref):
    st = st_ref[...]
    m = st[:, 0:1, :] / NN
    v = st[:, 1:2, :] / NN - m * m
    a = g_ref[...] * lax.rsqrt(v + EPS)
    cb = b_ref[...] - m * a
    hb = h_ref[...] * a + cb
    o = jnp.dot(hb[0], w2_ref[0], preferred_element_type=jnp.float32)
    for s in range(1, 4):
        o += jnp.dot(hb[s], w2_ref[s], preferred_element_type=jnp.float32)
    o *= dinv_ref[...]
    out_ref[0] = o[:, :32]
    out_ref[1] = o[:, 32:]


def _mid(h1, st1, g1, b1, w2r, dinv):
    return pl.pallas_call(
        _mid_body,
        grid=(NBLK,),
        in_specs=[
            pl.BlockSpec((4, BLK, 32), lambda i: (0, i, 0)),
            pl.BlockSpec((4, 8, 32), lambda i: (0, 0, 0)),
            pl.BlockSpec((4, 1, 32), lambda i: (0, 0, 0)),
            pl.BlockSpec((4, 1, 32), lambda i: (0, 0, 0)),
            pl.BlockSpec((4, 32, 64), lambda i: (0, 0, 0)),
            pl.BlockSpec((BLK, 1), lambda i: (i, 0)),
        ],
        out_specs=pl.BlockSpec((2, BLK, 32), lambda i: (0, i, 0)),
        out_shape=jax.ShapeDtypeStruct((2, NN, 32), jnp.float32),
    )(h1, st1, g1, b1, w2r, dinv)


def _pool_body(h_ref, st_ref, g_ref, b_ref, batch_ref, out_ref):
    i = pl.program_id(0)
    st = st_ref[...]
    m = st[:, 0:1, :] / NN
    v = st[:, 1:2, :] / NN - m * m
    a = g_ref[...] * lax.rsqrt(v + EPS)
    cb = b_ref[...] - m * a
    hb = h_ref[...] * a + cb
    bb = batch_ref[...].reshape(1, BLK)
    oh = (lax.broadcasted_iota(jnp.int32, (NG, BLK), 0)
          == jnp.broadcast_to(bb, (NG, BLK))).astype(jnp.float32)

    @pl.when(i == 0)
    def _():
        out_ref[...] = jnp.zeros_like(out_ref)

    out_ref[0] += jnp.dot(oh, hb[0], preferred_element_type=jnp.float32)
    out_ref[1] += jnp.dot(oh, hb[1], preferred_element_type=jnp.float32)


def _pool(h2, st2, g2, b2, batch3):
    return pl.pallas_call(
        _pool_body,
        grid=(NBLK,),
        in_specs=[
            pl.BlockSpec((2, BLK, 32), lambda i: (0, i, 0)),
            pl.BlockSpec((2, 8, 32), lambda i: (0, 0, 0)),
            pl.BlockSpec((2, 1, 32), lambda i: (0, 0, 0)),
            pl.BlockSpec((2, 1, 32), lambda i: (0, 0, 0)),
            pl.BlockSpec((1, 1, BLK), lambda i: (i, 0, 0)),
        ],
        out_specs=pl.BlockSpec((2, NG, 32), lambda i: (0, 0, 0)),
        out_shape=jax.ShapeDtypeStruct((2, NG, 32), jnp.float32),
    )(h2, st2, g2, b2, batch3)


def _head_body(p_ref, w1_ref, b1_ref, g3_ref, b3_ref, w2_ref, b2_ref,
               w3_ref, b3p_ref, out_ref):
    g0 = jnp.concatenate([p_ref[0], p_ref[1]], axis=1)
    g1 = jnp.maximum(
        jnp.dot(g0, w1_ref[...], preferred_element_type=jnp.float32)
        + b1_ref[...], 0.0)
    m = jnp.mean(g1, axis=0, keepdims=True)
    v = jnp.mean(g1 * g1, axis=0, keepdims=True) - m * m
    g1 = (g1 - m) * lax.rsqrt(v + EPS) * g3_ref[...] + b3_ref[...]
    g2 = jnp.maximum(
        jnp.dot(g1, w2_ref[...], preferred_element_type=jnp.float32)
        + b2_ref[...], 0.0)
    lg = jnp.dot(g2, w3_ref[...], preferred_element_type=jnp.float32) \
        + b3p_ref[...]
    mx = jnp.max(lg, axis=1, keepdims=True)
    lse = mx + jnp.log(jnp.sum(jnp.exp(lg - mx), axis=1, keepdims=True))
    out_ref[...] = lg - lse


def _head(pooled, w1, b1, g3, b3, w2, b2, w3p, b3p):
    return pl.pallas_call(
        _head_body,
        out_shape=jax.ShapeDtypeStruct((NG, 128), jnp.float32),
    )(pooled, w1, b1, g3, b3, w2, b2, w3p, b3p)


def kernel(x, edge_index, batch, W1, b1, bn1_g, bn1_b, W2, b2, bn2_g, bn2_b,
           fc1_W, fc1_b, bn3_g, bn3_b, fc2_W, fc2_b, fc3_W, fc3_b):
    src = edge_index[0].astype(jnp.int32)
    dst = edge_index[1].astype(jnp.int32)
    npad = EP - NE
    srcp = jnp.concatenate([src, jnp.zeros((npad,), jnp.int32)])
    dpad = NN + (jnp.arange(npad, dtype=jnp.int32) % (NP - NN))
    dstp = jnp.concatenate([dst, dpad])
    dst2 = dstp.reshape(EP // CH, CH)
    zeros1 = jnp.zeros((TPS,), jnp.float32)
    zeros2 = jnp.zeros((SUPE, 32), jnp.float32)

    degp = _deg_kernel(dst2, zeros1)
    p0 = degp[:NN].reshape(NN, 1)
    p1 = degp[NP:NP + NN].reshape(NN, 1)

    w1r = jnp.transpose(W1.reshape(NF, 4, 32), (1, 0, 2))
    hws1, dinv = _pre(x, w1r, p0, p1)
    acc1 = _msg1(hws1.reshape(4 * NN, 32), srcp, dst2, zeros2)
    acc1 = acc1.reshape(4, NP, 32)
    h1, st1 = _post1(acc1, hws1, dinv, b1.reshape(4, 1, 32))
    hws2 = _mid(h1, st1, bn1_g.reshape(4, 1, 32), bn1_b.reshape(4, 1, 32),
                W2.reshape(4, 32, 64), dinv)
    acc2 = _msg2(hws2.reshape(2 * NN, 32), srcp, dst2, zeros2)
    acc2 = acc2.reshape(2, NP, 32)
    h2, st2 = _post2(acc2, hws2, dinv, b2.reshape(2, 1, 32))
    batch3 = batch.astype(jnp.int32).reshape(NBLK, 1, BLK)
    pooled = _pool(h2, st2, bn2_g.reshape(2, 1, 32), bn2_b.reshape(2, 1, 32),
                   batch3)
    w3p = jnp.pad(fc3_W, ((0, 0), (0, 125)))
    b3p = jnp.concatenate(
        [fc3_b, jnp.full((125,), -1e30, jnp.float32)]).reshape(1, 128)
    out = _head(pooled, fc1_W, fc1_b.reshape(1, 64), bn3_g.reshape(1, 64),
                bn3_b.reshape(1, 64), fc2_W, fc2_b.reshape(1, 64), w3p, b3p)
    return out[:, :3]
